# Initial kernel scaffold; baseline (speedup 1.0000x reference)
import jax
import jax.numpy as jnp
from jax.experimental import pallas as pl


def kernel(x_user, x_item, edge_index_user_to_item, edge_index_item_to_user, l1_u2i_Wl, l1_u2i_bl, l1_u2i_Wr, l1_i2u_Wl, l1_i2u_bl, l1_i2u_Wr, l1_bn_user_g, l1_bn_user_b, l1_bn_item_g, l1_bn_item_b, l2_u2i_Wl, l2_u2i_bl, l2_u2i_Wr, l2_i2u_Wl, l2_i2u_bl, l2_i2u_Wr, l2_bn_user_g, l2_bn_user_b, l2_bn_item_g, l2_bn_item_b, l3_u2i_Wl, l3_u2i_bl, l3_u2i_Wr, l3_i2u_Wl, l3_i2u_bl, l3_i2u_Wr, l3_bn_user_g, l3_bn_user_b, l3_bn_item_g, l3_bn_item_b, post_user_W1, post_user_b1, post_user_bn_g, post_user_bn_b, post_user_W2, post_user_b2, post_item_W1, post_item_b1, post_item_bn_g, post_item_bn_b, post_item_W2, post_item_b2):
    raise NotImplementedError("write your pallas kernel here")


# speedup vs baseline: 3.0803x; 3.0803x over previous
import functools

import jax
import jax.numpy as jnp
from jax import lax
from jax.experimental import pallas as pl
from jax.experimental.pallas import tpu as pltpu
from jax.experimental.pallas import tpu_sc as plsc

N = 10000
D = 128
H = 64
E = 320000

NC = 2
NS = 16
CH = 128
EPT = 20480
NCH = EPT // CH
NP = 10016
NAC = 10048
TPT = NAC // NS
CW = 16

_f32 = jnp.float32


def _sc_agg_body(pa, pb, sa_h, da_h, sb_h, db_h,
                 out_a, out_b,
                 srcv, dstv, rows, obuf, acc):
    cid = lax.axis_index("c")
    sid = lax.axis_index("s")

    def _zrow(i, _):
        for j in range(H // 16):
            obuf[i, pl.ds(j * 16, 16)] = jnp.zeros((16,), _f32)
        return 0

    lax.fori_loop(0, TPT, _zrow, 0)
    pltpu.sync_copy(obuf, acc.at[pl.ds(sid * TPT, TPT)])
    plsc.subcore_barrier()

    def _do_type(p_hbm, s_hbm, d_hbm):
        pltpu.sync_copy(s_hbm.at[sid], srcv)
        pltpu.sync_copy(d_hbm.at[sid], dstv)

        def _step(j, _):
            pltpu.sync_copy(p_hbm.at[srcv.at[j]], rows)
            pltpu.sync_copy(rows, acc.at[dstv.at[j]], add=True)
            return 0

        lax.fori_loop(0, NCH, _step, 0)

    @pl.when(cid == 0)
    def _():
        _do_type(pa, sa_h, da_h)

    @pl.when(cid == 1)
    def _():
        _do_type(pb, sb_h, db_h)

    plsc.subcore_barrier()

    pltpu.sync_copy(acc.at[pl.ds(sid * TPT, TPT)], obuf)

    @pl.when(cid == 0)
    def _():
        pltpu.sync_copy(obuf, out_a.at[pl.ds(sid * TPT, TPT)])

    @pl.when(cid == 1)
    def _():
        pltpu.sync_copy(obuf, out_b.at[pl.ds(sid * TPT, TPT)])


_SC_MESH = plsc.VectorSubcoreMesh(
    core_axis_name="c", subcore_axis_name="s", num_cores=NC, num_subcores=NS)

_sc_agg = pl.kernel(
    _sc_agg_body,
    out_type=[jax.ShapeDtypeStruct((NAC, H), _f32)] * 2,
    mesh=_SC_MESH,
    scratch_types=[
        pltpu.VMEM((NCH, CH), jnp.int32),
        pltpu.VMEM((NCH, CH), jnp.int32),
        pltpu.VMEM((CH, H), _f32),
        pltpu.VMEM((TPT, H), _f32),
        pltpu.VMEM_SHARED((NAC, H), _f32),
    ],
    compiler_params=pltpu.CompilerParams(use_tc_tiling_on_sc=False),
    name="sc_edge_agg",
)


def _lrelu(x):
    return jnp.where(x >= 0, x, 0.01 * x)


def _bn(x, g, b):
    mu = jnp.mean(x, axis=0, keepdims=True)
    var = jnp.mean((x - mu) * (x - mu), axis=0, keepdims=True)
    return (x - mu) * lax.rsqrt(var + 1e-5) * g + b


def _store_padded(out_ref, val):
    out_ref[pl.ds(0, N), :] = val
    out_ref[pl.ds(N, NP - N), :] = jnp.zeros((NP - N, H), _f32)


def _tc_pre_body(xu, xi, wlu, wli, pu, pi):
    _store_padded(pu, jnp.dot(xu[:], wlu[:], preferred_element_type=_f32))
    _store_padded(pi, jnp.dot(xi[:], wli[:], preferred_element_type=_f32))


_tc_pre = pl.pallas_call(
    _tc_pre_body,
    out_shape=[jax.ShapeDtypeStruct((NP, H), _f32)] * 2,
)


def _combine_one(h_dst, s, c, Wr, bl, g, b, res):
    cnt = jnp.maximum(c[pl.ds(0, N), 0:1], 1.0)
    o = s[pl.ds(0, N), :] / cnt
    o = o + bl[:] + jnp.dot(h_dst[:], Wr[:], preferred_element_type=_f32)
    o = _lrelu(_bn(o, g[:], b[:]))
    if res:
        o = o + h_dst[:]
    return o


def _tc_combine_body(res, hu, hi, sa, sb, ca, cb,
                     Wr_u2i, bl_u2i, Wr_i2u, bl_i2u,
                     g_u, b_u, g_i, b_i,
                     nWl_u2i, nWl_i2u,
                     hu_out, hi_out, pu_out, pi_out):
    o_i = _combine_one(hi, sa, ca, Wr_u2i, bl_u2i, g_i, b_i, res)
    o_u = _combine_one(hu, sb, cb, Wr_i2u, bl_i2u, g_u, b_u, res)
    hu_out[:] = o_u
    hi_out[:] = o_i
    _store_padded(pu_out, jnp.dot(o_u, nWl_u2i[:], preferred_element_type=_f32))
    _store_padded(pi_out, jnp.dot(o_i, nWl_i2u[:], preferred_element_type=_f32))


def _make_tc_combine(res):
    return pl.pallas_call(
        functools.partial(_tc_combine_body, res),
        out_shape=[
            jax.ShapeDtypeStruct((N, H), _f32),
            jax.ShapeDtypeStruct((N, H), _f32),
            jax.ShapeDtypeStruct((NP, H), _f32),
            jax.ShapeDtypeStruct((NP, H), _f32),
        ],
    )


_tc_combine_l1 = _make_tc_combine(False)
_tc_combine_l2 = _make_tc_combine(True)


def _tc_post_body(hu, hi,
                  uW1, ub1, ug, ub, uW2, ub2,
                  iW1, ib1, ig, ib, iW2, ib2,
                  out_u, out_i):
    def _post(x, W1, b1, g, b, W2, b2):
        x = jnp.dot(x[:], W1[:], preferred_element_type=_f32) + b1[:]
        x = _lrelu(_bn(x, g[:], b[:]))
        return jnp.dot(x, W2[:], preferred_element_type=_f32) + b2[:]

    out_u[:] = _post(hu, uW1, ub1, ug, ub, uW2, ub2)
    out_i[:] = _post(hi, iW1, ib1, ig, ib, iW2, ib2)


_tc_post = pl.pallas_call(
    _tc_post_body,
    out_shape=[
        jax.ShapeDtypeStruct((N, H), _f32),
        jax.ShapeDtypeStruct((N, H), _f32),
    ],
)


def _prep_edges(ei):
    pad = NS * EPT - E
    src = jnp.concatenate([ei[0], jnp.full((pad,), N, jnp.int32)])
    dst = jnp.concatenate([ei[1], jnp.full((pad,), N, jnp.int32)])
    return src.reshape(NS, NCH, CH), dst.reshape(NS, NCH, CH)


def kernel(x_user, x_item, edge_index_user_to_item, edge_index_item_to_user, l1_u2i_Wl, l1_u2i_bl, l1_u2i_Wr, l1_i2u_Wl, l1_i2u_bl, l1_i2u_Wr, l1_bn_user_g, l1_bn_user_b, l1_bn_item_g, l1_bn_item_b, l2_u2i_Wl, l2_u2i_bl, l2_u2i_Wr, l2_i2u_Wl, l2_i2u_bl, l2_i2u_Wr, l2_bn_user_g, l2_bn_user_b, l2_bn_item_g, l2_bn_item_b, l3_u2i_Wl, l3_u2i_bl, l3_u2i_Wr, l3_i2u_Wl, l3_i2u_bl, l3_i2u_Wr, l3_bn_user_g, l3_bn_user_b, l3_bn_item_g, l3_bn_item_b, post_user_W1, post_user_b1, post_user_bn_g, post_user_bn_b, post_user_W2, post_user_b2, post_item_W1, post_item_b1, post_item_bn_g, post_item_bn_b, post_item_W2, post_item_b2):
    r2 = lambda v: v.reshape(1, H)
    sa_h, da_h = _prep_edges(edge_index_user_to_item)
    sb_h, db_h = _prep_edges(edge_index_item_to_user)

    ones_p = jnp.concatenate(
        [jnp.ones((N, H), _f32), jnp.zeros((NP - N, H), _f32)])
    ca, cb = _sc_agg(ones_p, ones_p, sa_h, da_h, sb_h, db_h)

    pu, pi = _tc_pre(x_user, x_item, l1_u2i_Wl, l1_i2u_Wl)
    sa, sb = _sc_agg(pu, pi, sa_h, da_h, sb_h, db_h)
    hu, hi, pu, pi = _tc_combine_l1(
        x_user, x_item, sa, sb, ca, cb,
        l1_u2i_Wr, r2(l1_u2i_bl), l1_i2u_Wr, r2(l1_i2u_bl),
        r2(l1_bn_user_g), r2(l1_bn_user_b), r2(l1_bn_item_g), r2(l1_bn_item_b),
        l2_u2i_Wl, l2_i2u_Wl)

    ws = (
        jnp.stack([l2_u2i_Wr, l3_u2i_Wr]),
        jnp.stack([r2(l2_u2i_bl), r2(l3_u2i_bl)]),
        jnp.stack([l2_i2u_Wr, l3_i2u_Wr]),
        jnp.stack([r2(l2_i2u_bl), r2(l3_i2u_bl)]),
        jnp.stack([r2(l2_bn_user_g), r2(l3_bn_user_g)]),
        jnp.stack([r2(l2_bn_user_b), r2(l3_bn_user_b)]),
        jnp.stack([r2(l2_bn_item_g), r2(l3_bn_item_g)]),
        jnp.stack([r2(l2_bn_item_b), r2(l3_bn_item_b)]),
        jnp.stack([l3_u2i_Wl, l3_u2i_Wl]),
        jnp.stack([l3_i2u_Wl, l3_i2u_Wl]),
    )

    def _layer(carry, w):
        hu, hi, pu, pi = carry
        (Wr_a, bl_a, Wr_b, bl_b, gu, bu, gi, bi, nWa, nWb) = w
        sa, sb = _sc_agg(pu, pi, sa_h, da_h, sb_h, db_h)
        hu, hi, pu, pi = _tc_combine_l2(
            hu, hi, sa, sb, ca, cb, Wr_a, bl_a, Wr_b, bl_b,
            gu, bu, gi, bi, nWa, nWb)
        return (hu, hi, pu, pi), None

    (hu, hi, pu, pi), _ = lax.scan(_layer, (hu, hi, pu, pi), ws)

    out_u, out_i = _tc_post(
        hu, hi,
        post_user_W1, r2(post_user_b1), r2(post_user_bn_g), r2(post_user_bn_b),
        post_user_W2, r2(post_user_b2),
        post_item_W1, r2(post_item_b1), r2(post_item_bn_g), r2(post_item_bn_b),
        post_item_W2, r2(post_item_b2))
    return out_u, out_i
